# baseline (speedup 1.0000x reference)
import functools

import jax
import jax.numpy as jnp
from jax import lax
from jax.experimental import pallas as pl
from jax.experimental.pallas import tpu as pltpu
from jax.experimental.pallas import tpu_sc as plsc

NUM_LOCATIONS = 100000
NUM_USERS = 100000
DIM = 64
BATCH = 4096
HIST = 50
HH = HIST // 2

NC, NS = 2, 16
NW = NC * NS
B_PER_NW = BATCH // NW

_mesh = plsc.VectorSubcoreMesh(core_axis_name="c", subcore_axis_name="s")


def _make_gather(nh, with_ts):
    out_type = [jax.ShapeDtypeStruct((nh, BATCH // 2, 2 * DIM), jnp.float32)]
    if with_ts:
        out_type.append(jax.ShapeDtypeStruct((24, DIM), jnp.float32))

    @functools.partial(
        pl.kernel,
        mesh=_mesh,
        out_type=out_type,
        scratch_types=[
            pltpu.VMEM((nh, B_PER_NW), jnp.int32),
            pltpu.VMEM((B_PER_NW, DIM), jnp.float32),
            pltpu.VMEM((B_PER_NW, DIM), jnp.float32),
            pltpu.VMEM((B_PER_NW, DIM), jnp.float32),
            pltpu.VMEM((B_PER_NW, DIM), jnp.float32),
            pltpu.SemaphoreType.DMA,
            pltpu.SemaphoreType.DMA,
            pltpu.SemaphoreType.DMA,
            pltpu.SemaphoreType.DMA,
            pltpu.SemaphoreType.DMA,
            pltpu.SemaphoreType.DMA,
            pltpu.SemaphoreType.DMA,
            pltpu.SemaphoreType.DMA,
            pltpu.SemaphoreType.DMA,
        ],
        compiler_params=pltpu.CompilerParams(use_tc_tiling_on_sc=False),
    )
    def _gather(idx_hbm, loc_tab, *rest):
        if with_ts:
            (ts_tab, loc_out, ts_out, idx_v, buf0, buf1, buf2, buf3,
             gsem0, gsem1, gsem2, gsem3,
             wsem0, wsem1, wsem2, wsem3, isem) = rest
        else:
            (loc_out, idx_v, buf0, buf1, buf2, buf3,
             gsem0, gsem1, gsem2, gsem3,
             wsem0, wsem1, wsem2, wsem3, isem) = rest
        wid = lax.axis_index("s") * NC + lax.axis_index("c")
        c0 = wid * B_PER_NW
        qq = (wid % (NW // 2)) * B_PER_NW
        d0 = (wid // (NW // 2)) * DIM

        K = 8
        for h0 in range(0, nh, K):
            hh = [pltpu.async_copy(
                      idx_hbm.at[pl.ds(h * BATCH + c0, B_PER_NW)],
                      idx_v.at[h], isem)
                  for h in range(h0, min(h0 + K, nh))]
            for hnd in hh:
                hnd.wait()

        bufs = (buf0, buf1, buf2, buf3)
        gsems = (gsem0, gsem1, gsem2, gsem3)
        wsems = (wsem0, wsem1, wsem2, wsem3)
        NBUF = 4

        def _write(h, buf, sem):
            return pltpu.async_copy(
                buf, loc_out.at[h, pl.ds(qq, B_PER_NW), pl.ds(d0, DIM)], sem)

        reads = [None] * NBUF
        writes = [None] * NBUF
        for t in range(nh + NBUF - 1):
            if t < nh:
                b = t % NBUF
                if writes[b] is not None:
                    writes[b].wait()
                reads[b] = pltpu.async_copy(
                    loc_tab.at[idx_v.at[t]], bufs[b], gsems[b])
            hp = t - (NBUF - 1)
            if 0 <= hp < nh:
                pb = hp % NBUF
                reads[pb].wait()
                writes[pb] = _write(hp, bufs[pb], wsems[pb])
        for k in range(NBUF):
            if writes[k] is not None:
                writes[k].wait()

        if with_ts:
            @pl.when(wid == 0)
            def _():
                pltpu.sync_copy(ts_tab, buf0.at[pl.ds(0, 24)])
                pltpu.sync_copy(buf0.at[pl.ds(0, 24)], ts_out)

    return _gather


_gather_a = _make_gather(HH, with_ts=True)
_gather_b = _make_gather(HIST - HH, with_ts=False)


def _copy_body(in_ref, out_ref):
    out_ref[...] = in_ref[...]


_COLS_PER_BLK = 6400
_user_copy_t = pl.pallas_call(
    _copy_body,
    grid=(NUM_USERS // _COLS_PER_BLK + 1,),
    in_specs=[pl.BlockSpec((DIM, _COLS_PER_BLK), lambda i: (0, i))],
    out_specs=pl.BlockSpec((DIM, _COLS_PER_BLK), lambda i: (0, i)),
    out_shape=jax.ShapeDtypeStruct((DIM, NUM_USERS), jnp.float32),
)


def _unpack_body(x_ref, y_ref):
    x = x_ref[0]
    y_ref[0] = jnp.concatenate([x[:, :DIM].T, x[:, DIM:].T], axis=1)


def _make_unpack(nh):
    return pl.pallas_call(
        _unpack_body,
        grid=(nh,),
        in_specs=[pl.BlockSpec((1, BATCH // 2, 2 * DIM),
                               lambda h: (h, 0, 0))],
        out_specs=pl.BlockSpec((1, DIM, BATCH), lambda h: (h, 0, 0)),
        out_shape=jax.ShapeDtypeStruct((nh, DIM, BATCH), jnp.float32),
    )


_unpack_a = _make_unpack(HH)
_unpack_b = _make_unpack(HIST - HH)


def kernel(location_x, location_table, user_table, timeslot_table):
    idx_t = location_x.T.reshape(BATCH * HIST).astype(jnp.int32)
    idx_a = lax.slice(idx_t, (0,), (HH * BATCH,))
    idx_b = lax.slice(idx_t, (HH * BATCH,), (HIST * BATCH,))
    loc_pa, ts = _gather_a(idx_a, location_table, timeslot_table)
    loc_pb, = _gather_b(idx_b, location_table)
    ya = _unpack_a(loc_pa)
    yb = _unpack_b(loc_pb)
    loc = jnp.transpose(jnp.concatenate([ya, yb], axis=0), (2, 0, 1))
    user = _user_copy_t(user_table.T).T
    return loc, ts, user


# speedup vs baseline: 1.1315x; 1.1315x over previous
import functools

import jax
import jax.numpy as jnp
from jax import lax
from jax.experimental import pallas as pl
from jax.experimental.pallas import tpu as pltpu
from jax.experimental.pallas import tpu_sc as plsc

NUM_LOCATIONS = 100000
NUM_USERS = 100000
DIM = 64
BATCH = 4096
HIST = 50
HH = HIST // 2

NC, NS = 2, 16
NW = NC * NS
B_PER_NW = BATCH // NW

_mesh = plsc.VectorSubcoreMesh(core_axis_name="c", subcore_axis_name="s")


def _make_gather(nh, with_ts):
    out_type = [jax.ShapeDtypeStruct((nh, BATCH // 2, 2 * DIM), jnp.float32)]
    if with_ts:
        out_type.append(jax.ShapeDtypeStruct((24, DIM), jnp.float32))

    @functools.partial(
        pl.kernel,
        mesh=_mesh,
        out_type=out_type,
        scratch_types=[
            pltpu.VMEM((nh, B_PER_NW), jnp.int32),
            pltpu.VMEM((B_PER_NW, DIM), jnp.float32),
            pltpu.VMEM((B_PER_NW, DIM), jnp.float32),
            pltpu.VMEM((B_PER_NW, DIM), jnp.float32),
            pltpu.VMEM((B_PER_NW, DIM), jnp.float32),
            pltpu.SemaphoreType.DMA,
            pltpu.SemaphoreType.DMA,
            pltpu.SemaphoreType.DMA,
            pltpu.SemaphoreType.DMA,
            pltpu.SemaphoreType.DMA,
            pltpu.SemaphoreType.DMA,
            pltpu.SemaphoreType.DMA,
            pltpu.SemaphoreType.DMA,
            pltpu.SemaphoreType.DMA,
        ],
        compiler_params=pltpu.CompilerParams(use_tc_tiling_on_sc=False),
    )
    def _gather(idx_hbm, loc_tab, *rest):
        if with_ts:
            (ts_tab, loc_out, ts_out, idx_v, buf0, buf1, buf2, buf3,
             gsem0, gsem1, gsem2, gsem3,
             wsem0, wsem1, wsem2, wsem3, isem) = rest
        else:
            (loc_out, idx_v, buf0, buf1, buf2, buf3,
             gsem0, gsem1, gsem2, gsem3,
             wsem0, wsem1, wsem2, wsem3, isem) = rest
        wid = lax.axis_index("s") * NC + lax.axis_index("c")
        c0 = wid * B_PER_NW
        qq = (wid % (NW // 2)) * B_PER_NW
        d0 = (wid // (NW // 2)) * DIM

        K = 8
        for h0 in range(0, nh, K):
            hh = [pltpu.async_copy(
                      idx_hbm.at[pl.ds(h * BATCH + c0, B_PER_NW)],
                      idx_v.at[h], isem)
                  for h in range(h0, min(h0 + K, nh))]
            for hnd in hh:
                hnd.wait()

        bufs = (buf0, buf1, buf2, buf3)
        gsems = (gsem0, gsem1, gsem2, gsem3)
        wsems = (wsem0, wsem1, wsem2, wsem3)
        NBUF = 4

        def _write(h, buf, sem):
            return pltpu.async_copy(
                buf, loc_out.at[h, pl.ds(qq, B_PER_NW), pl.ds(d0, DIM)], sem)

        reads = [None] * NBUF
        writes = [None] * NBUF
        for t in range(nh + NBUF - 1):
            if t < nh:
                b = t % NBUF
                if writes[b] is not None:
                    writes[b].wait()
                reads[b] = pltpu.async_copy(
                    loc_tab.at[idx_v.at[t]], bufs[b], gsems[b])
            hp = t - (NBUF - 1)
            if 0 <= hp < nh:
                pb = hp % NBUF
                reads[pb].wait()
                writes[pb] = _write(hp, bufs[pb], wsems[pb])
        for k in range(NBUF):
            if writes[k] is not None:
                writes[k].wait()

        if with_ts:
            @pl.when(wid == 0)
            def _():
                pltpu.sync_copy(ts_tab, buf0.at[pl.ds(0, 24)])
                pltpu.sync_copy(buf0.at[pl.ds(0, 24)], ts_out)

    return _gather


_gather_full = _make_gather(HIST, with_ts=True)


def _copy_body(in_ref, out_ref):
    out_ref[...] = in_ref[...]


_COLS_PER_BLK = 6400
_user_copy_t = pl.pallas_call(
    _copy_body,
    grid=(NUM_USERS // _COLS_PER_BLK + 1,),
    in_specs=[pl.BlockSpec((DIM, _COLS_PER_BLK), lambda i: (0, i))],
    out_specs=pl.BlockSpec((DIM, _COLS_PER_BLK), lambda i: (0, i)),
    out_shape=jax.ShapeDtypeStruct((DIM, NUM_USERS), jnp.float32),
)


def _unpack_body(x_ref, y_ref):
    x = x_ref[0]
    y_ref[0] = jnp.concatenate([x[:, :DIM].T, x[:, DIM:].T], axis=1)


def _make_unpack(nh):
    return pl.pallas_call(
        _unpack_body,
        grid=(nh,),
        in_specs=[pl.BlockSpec((1, BATCH // 2, 2 * DIM),
                               lambda h: (h, 0, 0))],
        out_specs=pl.BlockSpec((1, DIM, BATCH), lambda h: (h, 0, 0)),
        out_shape=jax.ShapeDtypeStruct((nh, DIM, BATCH), jnp.float32),
    )


_unpack_full = _make_unpack(HIST)


def kernel(location_x, location_table, user_table, timeslot_table):
    idx_t = location_x.T.reshape(BATCH * HIST).astype(jnp.int32)
    loc_p, ts = _gather_full(idx_t, location_table, timeslot_table)
    loc = jnp.transpose(_unpack_full(loc_p), (2, 0, 1))
    user = _user_copy_t(user_table.T).T
    return loc, ts, user


# speedup vs baseline: 1.1329x; 1.0013x over previous
import functools

import jax
import jax.numpy as jnp
from jax import lax
from jax.experimental import pallas as pl
from jax.experimental.pallas import tpu as pltpu
from jax.experimental.pallas import tpu_sc as plsc

NUM_LOCATIONS = 100000
NUM_USERS = 100000
DIM = 64
BATCH = 4096
HIST = 50

NC, NS = 2, 16
NW = NC * NS
B_PER_NW = BATCH // NW

_mesh = plsc.VectorSubcoreMesh(core_axis_name="c", subcore_axis_name="s")


def _make_gather(nh, with_ts):
    out_type = [jax.ShapeDtypeStruct((nh, BATCH // 2, 2 * DIM), jnp.float32)]
    if with_ts:
        out_type.append(jax.ShapeDtypeStruct((24, DIM), jnp.float32))

    @functools.partial(
        pl.kernel,
        mesh=_mesh,
        out_type=out_type,
        scratch_types=[
            pltpu.VMEM((nh, B_PER_NW), jnp.int32),
            pltpu.VMEM((B_PER_NW, DIM), jnp.float32),
            pltpu.VMEM((B_PER_NW, DIM), jnp.float32),
            pltpu.VMEM((B_PER_NW, DIM), jnp.float32),
            pltpu.VMEM((B_PER_NW, DIM), jnp.float32),
            pltpu.SemaphoreType.DMA,
            pltpu.SemaphoreType.DMA,
            pltpu.SemaphoreType.DMA,
            pltpu.SemaphoreType.DMA,
            pltpu.SemaphoreType.DMA,
            pltpu.SemaphoreType.DMA,
            pltpu.SemaphoreType.DMA,
            pltpu.SemaphoreType.DMA,
            pltpu.SemaphoreType.DMA,
        ],
        compiler_params=pltpu.CompilerParams(use_tc_tiling_on_sc=False),
    )
    def _gather(idx_hbm, loc_tab, *rest):
        if with_ts:
            (ts_tab, loc_out, ts_out, idx_v, buf0, buf1, buf2, buf3,
             gsem0, gsem1, gsem2, gsem3,
             wsem0, wsem1, wsem2, wsem3, isem) = rest
        else:
            (loc_out, idx_v, buf0, buf1, buf2, buf3,
             gsem0, gsem1, gsem2, gsem3,
             wsem0, wsem1, wsem2, wsem3, isem) = rest
        wid = lax.axis_index("s") * NC + lax.axis_index("c")
        c0 = wid * B_PER_NW
        qq = (wid % (NW // 2)) * B_PER_NW
        d0 = (wid // (NW // 2)) * DIM

        K = 8
        for h0 in range(0, nh, K):
            hh = [pltpu.async_copy(
                      idx_hbm.at[pl.ds(h * BATCH + c0, B_PER_NW)],
                      idx_v.at[h], isem)
                  for h in range(h0, min(h0 + K, nh))]
            for hnd in hh:
                hnd.wait()

        bufs = (buf0, buf1, buf2, buf3)
        gsems = (gsem0, gsem1, gsem2, gsem3)
        wsems = (wsem0, wsem1, wsem2, wsem3)
        NBUF = 4

        def _write(h, buf, sem):
            return pltpu.async_copy(
                buf, loc_out.at[h, pl.ds(qq, B_PER_NW), pl.ds(d0, DIM)], sem)

        reads = [None] * NBUF
        writes = [None] * NBUF
        for t in range(nh + NBUF - 1):
            if t < nh:
                b = t % NBUF
                if writes[b] is not None:
                    writes[b].wait()
                reads[b] = pltpu.async_copy(
                    loc_tab.at[idx_v.at[t]], bufs[b], gsems[b])
            hp = t - (NBUF - 1)
            if 0 <= hp < nh:
                pb = hp % NBUF
                reads[pb].wait()
                writes[pb] = _write(hp, bufs[pb], wsems[pb])
        for k in range(NBUF):
            if writes[k] is not None:
                writes[k].wait()

        if with_ts:
            @pl.when(wid == 0)
            def _():
                pltpu.sync_copy(ts_tab, buf0.at[pl.ds(0, 24)])
                pltpu.sync_copy(buf0.at[pl.ds(0, 24)], ts_out)

    return _gather


_gather_full = _make_gather(HIST, with_ts=True)


def _copy_body(in_ref, out_ref):
    out_ref[...] = in_ref[...]


_COLS_PER_BLK = 6400
_user_copy_t = pl.pallas_call(
    _copy_body,
    grid=(NUM_USERS // _COLS_PER_BLK + 1,),
    in_specs=[pl.BlockSpec((DIM, _COLS_PER_BLK), lambda i: (0, i))],
    out_specs=pl.BlockSpec((DIM, _COLS_PER_BLK), lambda i: (0, i)),
    out_shape=jax.ShapeDtypeStruct((DIM, NUM_USERS), jnp.float32),
)


def _unpack_body(x_ref, y_ref):
    x = x_ref[0]
    y_ref[0] = jnp.concatenate([x[:, :DIM].T, x[:, DIM:].T], axis=1)


def _make_unpack(nh):
    return pl.pallas_call(
        _unpack_body,
        grid=(nh,),
        in_specs=[pl.BlockSpec((1, BATCH // 2, 2 * DIM),
                               lambda h: (h, 0, 0))],
        out_specs=pl.BlockSpec((1, DIM, BATCH), lambda h: (h, 0, 0)),
        out_shape=jax.ShapeDtypeStruct((nh, DIM, BATCH), jnp.float32),
    )


_unpack_full = _make_unpack(HIST)


def kernel(location_x, location_table, user_table, timeslot_table):
    idx_t = location_x.T.reshape(BATCH * HIST).astype(jnp.int32)
    loc_p, ts = _gather_full(idx_t, location_table, timeslot_table)
    loc = jnp.transpose(_unpack_full(loc_p), (2, 0, 1))
    user = _user_copy_t(user_table.T).T
    return loc, ts, user
